# baseline (speedup 1.0000x reference)
import functools

import jax
import jax.numpy as jnp
from jax import lax
from jax.experimental import pallas as pl
from jax.experimental.pallas import tpu as pltpu
from jax.experimental.pallas import tpu_sc as plsc

_HID = 768
_NEXP = 8
_LANES = 16
_NCORES = 2
_NSUB = 16
_NWORK = _NCORES * _NSUB
_TBLK = 128

_TC_BLK = 8192


def _logits_body(x_ref, w_ref, out_ref):
    logits = lax.dot_general(
        x_ref[...], w_ref[...],
        dimension_numbers=(((1,), (1,)), ((), ())),
        preferred_element_type=jnp.float32)
    t = jnp.transpose(logits)
    nblk = _TC_BLK // _TBLK
    out_ref[...] = jnp.transpose(
        t.reshape(_NEXP, nblk, _TBLK), (1, 0, 2))


def _compute_logits_t3(flat, w_t):
    n_tok = flat.shape[0]
    nblk = _TC_BLK // _TBLK
    return pl.pallas_call(
        _logits_body,
        grid=(n_tok // _TC_BLK,),
        in_specs=[
            pl.BlockSpec((_TC_BLK, _HID), lambda i: (i, 0)),
            pl.BlockSpec((_NEXP, _HID), lambda i: (0, 0)),
        ],
        out_specs=pl.BlockSpec((nblk, _NEXP, _TBLK), lambda i: (i, 0, 0)),
        out_shape=jax.ShapeDtypeStruct((n_tok // _TBLK, _NEXP, _TBLK),
                                       jnp.float32),
    )(flat, w_t)


def _make_router(n_tok):
    tpw = n_tok // _NWORK
    bpw = tpw // _TBLK

    @functools.partial(
        pl.kernel,
        mesh=plsc.VectorSubcoreMesh(core_axis_name="c", subcore_axis_name="s"),
        compiler_params=pltpu.CompilerParams(needs_layout_passes=False,
                                             skip_device_barrier=True),
        out_type=[
            jax.ShapeDtypeStruct((n_tok * 2,), jnp.float32),
            jax.ShapeDtypeStruct((n_tok * 2,), jnp.int32),
        ],
        scratch_types=[
            pltpu.VMEM((tpw * _NEXP,), jnp.float32),
            pltpu.VMEM((tpw * 2,), jnp.float32),
            pltpu.VMEM((tpw * 2,), jnp.int32),
        ],
    )
    def router(logits_hbm, w_hbm, i_hbm, lg_v, w_v, i_v):
        wid = lax.axis_index("s") * _NCORES + lax.axis_index("c")
        pltpu.sync_copy(logits_hbm.at[pl.ds(wid * tpw * _NEXP, tpw * _NEXP)],
                        lg_v)

        def group(k, carry):
            b = k >> 3
            g = k & 7
            lbase = b * (_TBLK * _NEXP) + g * _LANES
            ls = [lg_v[pl.ds(lbase + e * _TBLK, _LANES)] for e in range(_NEXP)]
            big = jnp.full((_LANES,), _NEXP, jnp.int32)
            m1 = ls[0]
            for e in range(1, _NEXP):
                m1 = jnp.maximum(m1, ls[e])
            i1 = big
            for e in range(_NEXP - 1, -1, -1):
                i1 = jnp.where(ls[e] == m1, e, i1)
            neg = jnp.full((_LANES,), -jnp.inf, jnp.float32)
            ms = [jnp.where(i1 == e, neg, ls[e]) for e in range(_NEXP)]
            m2 = ms[0]
            for e in range(1, _NEXP):
                m2 = jnp.maximum(m2, ms[e])
            i2 = big
            for e in range(_NEXP - 1, -1, -1):
                i2 = jnp.where(ms[e] == m2, e, i2)
            z = jnp.exp(m2 - m1)
            s = 1.0 + z
            w1 = 1.0 / s
            w2 = z / s
            obase = b * (_TBLK * 2) + g * _LANES
            w_v[pl.ds(obase, _LANES)] = w1
            w_v[pl.ds(obase + _TBLK, _LANES)] = w2
            i_v[pl.ds(obase, _LANES)] = i1
            i_v[pl.ds(obase + _TBLK, _LANES)] = i2
            return carry

        lax.fori_loop(0, bpw * (_TBLK // _LANES), group, 0)
        pltpu.sync_copy(w_v, w_hbm.at[pl.ds(wid * tpw * 2, tpw * 2)])
        pltpu.sync_copy(i_v, i_hbm.at[pl.ds(wid * tpw * 2, tpw * 2)])

    return router


def kernel(hidden_states, gate_w):
    b, s, h = hidden_states.shape
    flat = hidden_states.reshape(-1, h)
    n_tok = flat.shape[0]
    nblk = n_tok // _TBLK
    logits_t3 = _compute_logits_t3(flat, gate_w)
    w_flat, i_flat = _make_router(n_tok)(logits_t3.reshape(-1))
    weights = w_flat.reshape(nblk, 2, _TBLK).transpose(0, 2, 1)
    weights = weights.reshape(n_tok, 2)
    indices = i_flat.reshape(nblk, 2, _TBLK).transpose(0, 2, 1)
    indices = indices.reshape(n_tok, 2)
    logits = logits_t3.transpose(0, 2, 1).reshape(n_tok, _NEXP)
    return weights, indices, logits


# speedup vs baseline: 1.0526x; 1.0526x over previous
import functools

import jax
import jax.numpy as jnp
from jax import lax
from jax.experimental import pallas as pl
from jax.experimental.pallas import tpu as pltpu
from jax.experimental.pallas import tpu_sc as plsc

_HID = 768
_NEXP = 8
_LANES = 16
_NCORES = 2
_NSUB = 16
_NWORK = _NCORES * _NSUB
_TBLK = 128

_TC_BLK = 4096


def _logits_body(x_ref, w_ref, out_ref):
    logits = lax.dot_general(
        x_ref[...], w_ref[...],
        dimension_numbers=(((1,), (1,)), ((), ())),
        preferred_element_type=jnp.float32)
    t = jnp.transpose(logits)
    nblk = _TC_BLK // _TBLK
    out_ref[...] = jnp.transpose(
        t.reshape(_NEXP, nblk, _TBLK), (1, 0, 2))


def _compute_logits_t3(flat, w_t):
    n_tok = flat.shape[0]
    nblk = _TC_BLK // _TBLK
    return pl.pallas_call(
        _logits_body,
        grid=(n_tok // _TC_BLK,),
        in_specs=[
            pl.BlockSpec((_TC_BLK, _HID), lambda i: (i, 0)),
            pl.BlockSpec((_NEXP, _HID), lambda i: (0, 0)),
        ],
        out_specs=pl.BlockSpec((nblk, _NEXP, _TBLK), lambda i: (i, 0, 0)),
        out_shape=jax.ShapeDtypeStruct((n_tok // _TBLK, _NEXP, _TBLK),
                                       jnp.float32),
    )(flat, w_t)


def _make_router(n_tok):
    tpw = n_tok // _NWORK
    bpw = tpw // _TBLK

    @functools.partial(
        pl.kernel,
        mesh=plsc.VectorSubcoreMesh(core_axis_name="c", subcore_axis_name="s"),
        compiler_params=pltpu.CompilerParams(needs_layout_passes=False,
                                             skip_device_barrier=True),
        out_type=[
            jax.ShapeDtypeStruct((n_tok * 2,), jnp.float32),
            jax.ShapeDtypeStruct((n_tok * 2,), jnp.int32),
        ],
        scratch_types=[
            pltpu.VMEM((tpw * _NEXP,), jnp.float32),
            pltpu.VMEM((tpw * 2,), jnp.float32),
            pltpu.VMEM((tpw * 2,), jnp.int32),
        ],
    )
    def router(logits_hbm, w_hbm, i_hbm, lg_v, w_v, i_v):
        wid = lax.axis_index("s") * _NCORES + lax.axis_index("c")
        pltpu.sync_copy(logits_hbm.at[pl.ds(wid * tpw * _NEXP, tpw * _NEXP)],
                        lg_v)

        def group(k, carry):
            b = k >> 3
            g = k & 7
            lbase = b * (_TBLK * _NEXP) + g * _LANES
            ls = [lg_v[pl.ds(lbase + e * _TBLK, _LANES)] for e in range(_NEXP)]
            big = jnp.full((_LANES,), _NEXP, jnp.int32)
            m1 = ls[0]
            for e in range(1, _NEXP):
                m1 = jnp.maximum(m1, ls[e])
            i1 = big
            for e in range(_NEXP - 1, -1, -1):
                i1 = jnp.where(ls[e] == m1, e, i1)
            neg = jnp.full((_LANES,), -jnp.inf, jnp.float32)
            ms = [jnp.where(i1 == e, neg, ls[e]) for e in range(_NEXP)]
            m2 = ms[0]
            for e in range(1, _NEXP):
                m2 = jnp.maximum(m2, ms[e])
            i2 = big
            for e in range(_NEXP - 1, -1, -1):
                i2 = jnp.where(ms[e] == m2, e, i2)
            z = jnp.exp(m2 - m1)
            s = 1.0 + z
            w1 = 1.0 / s
            w2 = z / s
            obase = b * (_TBLK * 2) + g * _LANES
            w_v[pl.ds(obase, _LANES)] = w1
            w_v[pl.ds(obase + _TBLK, _LANES)] = w2
            i_v[pl.ds(obase, _LANES)] = i1
            i_v[pl.ds(obase + _TBLK, _LANES)] = i2
            return carry

        lax.fori_loop(0, bpw * (_TBLK // _LANES), group, 0)
        pltpu.sync_copy(w_v, w_hbm.at[pl.ds(wid * tpw * 2, tpw * 2)])
        pltpu.sync_copy(i_v, i_hbm.at[pl.ds(wid * tpw * 2, tpw * 2)])

    return router


def kernel(hidden_states, gate_w):
    b, s, h = hidden_states.shape
    flat = hidden_states.reshape(-1, h)
    n_tok = flat.shape[0]
    nblk = n_tok // _TBLK
    logits_t3 = _compute_logits_t3(flat, gate_w)
    w_flat, i_flat = _make_router(n_tok)(logits_t3.reshape(-1))
    weights = w_flat.reshape(nblk, 2, _TBLK).transpose(0, 2, 1)
    weights = weights.reshape(n_tok, 2)
    indices = i_flat.reshape(nblk, 2, _TBLK).transpose(0, 2, 1)
    indices = indices.reshape(n_tok, 2)
    logits = logits_t3.transpose(0, 2, 1).reshape(n_tok, _NEXP)
    return weights, indices, logits
